# speedup vs baseline: 1.5562x; 1.5562x over previous
import jax
import jax.numpy as jnp
from jax.experimental import pallas as pl
from jax.experimental.pallas import tpu as pltpu

B = 16384
F_IN = 768
H = 1024
BM = 512


def _fused_kernel(xs_ref, xn_ref, w_ref, b_ref, ow_ref, ob_ref, o_ref):
    w = w_ref[...]
    b = b_ref[...]
    acc_s = jnp.dot(xs_ref[...], w, preferred_element_type=jnp.float32) + b
    acc_n = jnp.dot(xn_ref[...], w, preferred_element_type=jnp.float32) + b
    hs = jnp.square(jnp.clip(acc_s, 0.0, 1.0))
    hn = jnp.square(jnp.clip(acc_n, 0.0, 1.0))
    contrib = hs * ow_ref[0:1, :] + hn * ow_ref[1:2, :]
    logit = jnp.sum(contrib, axis=1, keepdims=True) + ob_ref[0, 0]
    o_ref[...] = jax.nn.sigmoid(logit)


def kernel(stm_dense, nstm_dense, ft_w, ft_b, out_w, out_b):
    wt = ft_w.T
    b2 = ft_b.reshape(1, H)
    ow2 = out_w.reshape(2, H)
    ob2 = out_b.reshape(1, 1)
    grid = (B // BM,)
    return pl.pallas_call(
        _fused_kernel,
        grid=grid,
        in_specs=[
            pl.BlockSpec((BM, F_IN), lambda i: (i, 0)),
            pl.BlockSpec((BM, F_IN), lambda i: (i, 0)),
            pl.BlockSpec((F_IN, H), lambda i: (0, 0)),
            pl.BlockSpec((1, H), lambda i: (0, 0)),
            pl.BlockSpec((2, H), lambda i: (0, 0)),
            pl.BlockSpec((1, 1), lambda i: (0, 0)),
        ],
        out_specs=pl.BlockSpec((BM, 1), lambda i: (i, 0)),
        out_shape=jax.ShapeDtypeStruct((B, 1), jnp.float32),
        compiler_params=pltpu.CompilerParams(
            dimension_semantics=("parallel",),
        ),
    )(stm_dense, nstm_dense, wt, b2, ow2, ob2)


# speedup vs baseline: 1.5834x; 1.0175x over previous
import jax
import jax.numpy as jnp
from jax.experimental import pallas as pl
from jax.experimental.pallas import tpu as pltpu

B = 16384
F_IN = 768
H = 1024
BM = 512


def _fused_kernel(xs_ref, xn_ref, w_ref, b_ref, ow_ref, ob_ref, o_ref):
    w = w_ref[...]
    b = b_ref[...]
    xs = xs_ref[...].astype(jnp.bfloat16)
    xn = xn_ref[...].astype(jnp.bfloat16)
    acc_s = jnp.dot(xs, w, preferred_element_type=jnp.float32) + b
    acc_n = jnp.dot(xn, w, preferred_element_type=jnp.float32) + b
    hs = jnp.square(jnp.clip(acc_s, 0.0, 1.0))
    hn = jnp.square(jnp.clip(acc_n, 0.0, 1.0))
    contrib = hs * ow_ref[0:1, :] + hn * ow_ref[1:2, :]
    logit = jnp.sum(contrib, axis=1, keepdims=True) + ob_ref[0, 0]
    o_ref[...] = jax.nn.sigmoid(logit)


def kernel(stm_dense, nstm_dense, ft_w, ft_b, out_w, out_b):
    wt = ft_w.T.astype(jnp.bfloat16)
    b2 = ft_b.reshape(1, H)
    ow2 = out_w.reshape(2, H)
    ob2 = out_b.reshape(1, 1)
    grid = (B // BM,)
    return pl.pallas_call(
        _fused_kernel,
        grid=grid,
        in_specs=[
            pl.BlockSpec((BM, F_IN), lambda i: (i, 0)),
            pl.BlockSpec((BM, F_IN), lambda i: (i, 0)),
            pl.BlockSpec((F_IN, H), lambda i: (0, 0)),
            pl.BlockSpec((1, H), lambda i: (0, 0)),
            pl.BlockSpec((2, H), lambda i: (0, 0)),
            pl.BlockSpec((1, 1), lambda i: (0, 0)),
        ],
        out_specs=pl.BlockSpec((BM, 1), lambda i: (i, 0)),
        out_shape=jax.ShapeDtypeStruct((B, 1), jnp.float32),
        compiler_params=pltpu.CompilerParams(
            dimension_semantics=("parallel",),
        ),
    )(stm_dense, nstm_dense, wt, b2, ow2, ob2)


# speedup vs baseline: 1.6011x; 1.0112x over previous
import jax
import jax.numpy as jnp
from jax.experimental import pallas as pl
from jax.experimental.pallas import tpu as pltpu

B = 16384
F_IN = 768
H = 1024
BM = 512

_DN = (((1,), (1,)), ((), ()))


def _fused_kernel(xs_ref, xn_ref, w_ref, b_ref, ow_ref, ob_ref, o_ref):
    w = w_ref[...]
    b = b_ref[...]
    acc_s = jax.lax.dot_general(xs_ref[...], w, _DN,
                                preferred_element_type=jnp.float32) + b
    acc_n = jax.lax.dot_general(xn_ref[...], w, _DN,
                                preferred_element_type=jnp.float32) + b
    hs = jnp.square(jnp.clip(acc_s, 0.0, 1.0))
    hn = jnp.square(jnp.clip(acc_n, 0.0, 1.0))
    contrib = hs * ow_ref[0:1, :] + hn * ow_ref[1:2, :]
    logit = jnp.sum(contrib, axis=1, keepdims=True) + ob_ref[0, 0]
    o_ref[...] = jax.nn.sigmoid(logit)


def kernel(stm_dense, nstm_dense, ft_w, ft_b, out_w, out_b):
    b2 = ft_b.reshape(1, H)
    ow2 = out_w.reshape(2, H)
    ob2 = out_b.reshape(1, 1)
    grid = (B // BM,)
    return pl.pallas_call(
        _fused_kernel,
        grid=grid,
        in_specs=[
            pl.BlockSpec((BM, F_IN), lambda i: (i, 0)),
            pl.BlockSpec((BM, F_IN), lambda i: (i, 0)),
            pl.BlockSpec((H, F_IN), lambda i: (0, 0)),
            pl.BlockSpec((1, H), lambda i: (0, 0)),
            pl.BlockSpec((2, H), lambda i: (0, 0)),
            pl.BlockSpec((1, 1), lambda i: (0, 0)),
        ],
        out_specs=pl.BlockSpec((BM, 1), lambda i: (i, 0)),
        out_shape=jax.ShapeDtypeStruct((B, 1), jnp.float32),
        compiler_params=pltpu.CompilerParams(
            dimension_semantics=("parallel",),
        ),
    )(stm_dense, nstm_dense, ft_w, b2, ow2, ob2)


# speedup vs baseline: 1.6933x; 1.0576x over previous
import jax
import jax.numpy as jnp
from jax.experimental import pallas as pl
from jax.experimental.pallas import tpu as pltpu

B = 16384
F_IN = 768
H = 1024
BM = 1024

_DN = (((1,), (1,)), ((), ()))


def _fused_kernel(xs_ref, xn_ref, w_ref, b_ref, ow_ref, ob_ref, o_ref):
    w = w_ref[...]
    b = b_ref[...]
    acc_s = jax.lax.dot_general(xs_ref[...], w, _DN,
                                preferred_element_type=jnp.float32) + b
    acc_n = jax.lax.dot_general(xn_ref[...], w, _DN,
                                preferred_element_type=jnp.float32) + b
    hs = jnp.square(jnp.clip(acc_s, 0.0, 1.0))
    hn = jnp.square(jnp.clip(acc_n, 0.0, 1.0))
    contrib = hs * ow_ref[0:1, :] + hn * ow_ref[1:2, :]
    logit = jnp.sum(contrib, axis=1, keepdims=True) + ob_ref[0, 0]
    o_ref[...] = jax.nn.sigmoid(logit)


def kernel(stm_dense, nstm_dense, ft_w, ft_b, out_w, out_b):
    b2 = ft_b.reshape(1, H)
    ow2 = out_w.reshape(2, H)
    ob2 = out_b.reshape(1, 1)
    grid = (B // BM,)
    return pl.pallas_call(
        _fused_kernel,
        grid=grid,
        in_specs=[
            pl.BlockSpec((BM, F_IN), lambda i: (i, 0)),
            pl.BlockSpec((BM, F_IN), lambda i: (i, 0)),
            pl.BlockSpec((H, F_IN), lambda i: (0, 0)),
            pl.BlockSpec((1, H), lambda i: (0, 0)),
            pl.BlockSpec((2, H), lambda i: (0, 0)),
            pl.BlockSpec((1, 1), lambda i: (0, 0)),
        ],
        out_specs=pl.BlockSpec((BM, 1), lambda i: (i, 0)),
        out_shape=jax.ShapeDtypeStruct((B, 1), jnp.float32),
        compiler_params=pltpu.CompilerParams(
            dimension_semantics=("parallel",),
        ),
    )(stm_dense, nstm_dense, ft_w, b2, ow2, ob2)
